# baseline (speedup 1.0000x reference)
import functools

import jax
import jax.numpy as jnp
from jax import lax
from jax.experimental import pallas as pl
from jax.experimental.pallas import tpu as pltpu
from jax.experimental.pallas import tpu_sc as plsc

T = 2048
D = 1024
F = 1408
E = 8
K = 2

N = T * K
BTG = 128
NPAD = N + E * BTG
NBLK = NPAD // BTG
NC = 2
NS = 16
L = 16
CHUNK = N // (NC * NS)
SCAN = N // NS

_MESH = dict(core_axis_name="c", subcore_axis_name="s", num_cores=NC,
             num_subcores=NS)


def _router_body(x_ref, gate_ref, topi_ref, topw_ref):
    logits = lax.dot_general(x_ref[...], gate_ref[...],
                             (((1,), (1,)), ((), ())),
                             preferred_element_type=jnp.float32)
    m = jnp.max(logits, axis=1, keepdims=True)
    ex = jnp.exp(logits - m)
    probs = ex / jnp.sum(ex, axis=1, keepdims=True)
    iota_e = lax.broadcasted_iota(jnp.int32, probs.shape, 1)
    m1 = jnp.max(probs, axis=1, keepdims=True)
    i1 = jnp.min(jnp.where(probs == m1, iota_e, E), axis=1, keepdims=True)
    masked = jnp.where(iota_e == i1, -jnp.inf, probs)
    m2 = jnp.max(masked, axis=1, keepdims=True)
    i2 = jnp.min(jnp.where(masked == m2, iota_e, E), axis=1, keepdims=True)
    topi_ref[...] = jnp.concatenate([i1, i2], axis=1)
    topw_ref[...] = jnp.concatenate([m1, m2], axis=1)


def _router(x, gate_w):
    return pl.pallas_call(
        _router_body,
        out_shape=(jax.ShapeDtypeStruct((T, K), jnp.int32),
                   jax.ShapeDtypeStruct((T, K), jnp.float32)),
    )(x, gate_w)


def _lane_iota():
    return lax.broadcasted_iota(jnp.int32, (L,), 0)


_DNUMS = lax.GatherDimensionNumbers(
    offset_dims=(), collapsed_slice_dims=(0,), start_index_map=(0,))


def _take(v, idx):
    return lax.gather(v, idx[:, None], _DNUMS, slice_sizes=(1,),
                      mode=lax.GatherScatterMode.PROMISE_IN_BOUNDS)


def _vsum(v):
    ln = _lane_iota()
    for k in (1, 2, 4, 8):
        v = v + _take(v, jnp.bitwise_xor(ln, k))
    return v


def _vcumsum(v):
    ln = _lane_iota()
    for k in (1, 2, 4, 8):
        shifted = _take(v, jnp.maximum(ln - k, 0))
        v = v + jnp.where(ln >= k, shifted, jnp.zeros_like(v))
    return v


def _lane_splat(vec, lane):
    return _vsum(jnp.where(_lane_iota() == lane, vec, jnp.zeros_like(vec)))


def _dispatch_body(eids_hbm, x_hbm, pos_hbm, blkmap_hbm, valid_hbm, xs_hbm,
                   ids_v, cnt_v, allcnt_v, pos_b, tok_b, xbuf_v,
                   shared_cnt, gsem, ssem):
    cc = lax.axis_index("c")
    sid = lax.axis_index("s")
    lanes = _lane_iota()

    pltpu.sync_copy(eids_hbm.at[pl.ds(sid * SCAN, SCAN)], ids_v)
    counts = jnp.zeros((L,), jnp.int32)
    for v in range(SCAN // L):
        ids = ids_v[pl.ds(v * L, L)]
        for e in range(E):
            pc = _vsum(jnp.where(ids == e, 1, 0))
            counts = counts + jnp.where(lanes == e, pc, 0)
    cnt_v[...] = counts
    pltpu.sync_copy(cnt_v, shared_cnt.at[pl.ds(sid * L, L)])
    plsc.subcore_barrier()

    pltpu.sync_copy(shared_cnt, allcnt_v)
    totals = jnp.zeros((L,), jnp.int32)
    prefix = jnp.zeros((L,), jnp.int32)
    for s in range(NS):
        row = allcnt_v[pl.ds(s * L, L)]
        totals = totals + row
        prefix = prefix + jnp.where(s < sid, row, 0)
    asize = jnp.where(lanes < E,
                      jnp.bitwise_and(totals + (BTG - 1), -BTG), 0)
    cum_asize = _vcumsum(asize)
    aoff = cum_asize - asize
    run = aoff + prefix
    total_pad = _lane_splat(cum_asize, E - 1)

    ends = [_lane_splat(cum_asize, e) for e in range(E)]

    @pl.when(jnp.logical_and(cc == 0, sid == 0))
    def _():
        for b3 in range(3):
            bs = (b3 * L + lanes) * BTG
            eid = jnp.zeros((L,), jnp.int32)
            for e in range(E - 1):
                eid = eid + jnp.where(bs >= ends[e], 1, 0)
            cnt_v[...] = eid
            pltpu.sync_copy(cnt_v, blkmap_hbm.at[pl.ds(b3 * L, L)])
            cnt_v[...] = jnp.where(bs < total_pad, 1, 0)
            pltpu.sync_copy(cnt_v, valid_hbm.at[pl.ds(b3 * L, L)])

    for v in range(SCAN // L):
        ids = ids_v[pl.ds(v * L, L)]
        pos = jnp.zeros((L,), jnp.int32)
        for e in range(E):
            msk = ids == e
            mi = jnp.where(msk, 1, 0)
            cum = _vcumsum(mi)
            s_e = _lane_splat(run, e)
            pos = jnp.where(msk, s_e + cum - 1, pos)
            run = run + jnp.where(lanes == e, _vsum(mi), 0)
        tok = (sid * SCAN + v * L + lanes) >> 1
        j, half = (v % 8) // 2, (v % 2) * L

        @pl.when(cc == (0 if v < 8 else 1))
        def _(pos=pos, tok=tok, j=j, half=half):
            pos_b[j][pl.ds(half, L)] = pos
            tok_b[j][pl.ds(half, L)] = tok

    rbase = NS // 2 * sid + 4 * cc
    for j in range(4):
        pltpu.sync_copy(pos_b[j], pos_hbm.at[rbase + j])
        pltpu.async_copy(x_hbm.at[tok_b[j]], xbuf_v, gsem).wait()
        pltpu.async_copy(xbuf_v, xs_hbm.at[pos_b[j]], ssem).wait()


def _dispatch(eids, x):
    kfn = pl.kernel(
        _dispatch_body,
        out_type=(jax.ShapeDtypeStruct((N // 32, 32), jnp.int32),
                  jax.ShapeDtypeStruct((48,), jnp.int32),
                  jax.ShapeDtypeStruct((48,), jnp.int32),
                  jax.ShapeDtypeStruct((NPAD, D), jnp.float32)),
        mesh=plsc.VectorSubcoreMesh(**_MESH),
        scratch_types=[
            pltpu.VMEM((SCAN,), jnp.int32),
            pltpu.VMEM((L,), jnp.int32),
            pltpu.VMEM((NS * L,), jnp.int32),
            tuple(pltpu.VMEM((32,), jnp.int32) for _ in range(4)),
            tuple(pltpu.VMEM((32,), jnp.int32) for _ in range(4)),
            pltpu.VMEM((32, D), jnp.float32),
            pltpu.VMEM_SHARED((NS * L,), jnp.int32),
            pltpu.SemaphoreType.DMA,
            pltpu.SemaphoreType.DMA,
        ],
    )
    return kfn(eids, x)


def _gemm_body(bm_ref, valid_ref, xs_ref, w1_ref, w1u_ref, w2_ref, y_ref):
    s = pl.program_id(0)

    @pl.when(valid_ref[s] != 0)
    def _():
        xb = xs_ref[...]
        h = lax.dot_general(xb, w1_ref[0], (((1,), (1,)), ((), ())),
                            preferred_element_type=jnp.float32)
        u = lax.dot_general(xb, w1u_ref[0], (((1,), (1,)), ((), ())),
                            preferred_element_type=jnp.float32)
        g = h * (1.0 / (1.0 + jnp.exp(-h))) * u
        y_ref[...] = lax.dot_general(g, w2_ref[0], (((1,), (1,)), ((), ())),
                                     preferred_element_type=jnp.float32)


def _gemm(blkmap, valid, xs, w1, w1_up, w2):
    grid_spec = pltpu.PrefetchScalarGridSpec(
        num_scalar_prefetch=2,
        grid=(NBLK,),
        in_specs=[
            pl.BlockSpec((BTG, D), lambda s, bm, vd: (s, 0)),
            pl.BlockSpec((1, F, D), lambda s, bm, vd: (bm[s], 0, 0)),
            pl.BlockSpec((1, F, D), lambda s, bm, vd: (bm[s], 0, 0)),
            pl.BlockSpec((1, D, F), lambda s, bm, vd: (bm[s], 0, 0)),
        ],
        out_specs=pl.BlockSpec((BTG, D), lambda s, bm, vd: (s, 0)),
    )
    return pl.pallas_call(
        _gemm_body,
        grid_spec=grid_spec,
        out_shape=jax.ShapeDtypeStruct((NPAD, D), jnp.float32),
        compiler_params=pltpu.CompilerParams(
            dimension_semantics=("arbitrary",),
        ),
    )(blkmap, valid, xs, w1, w1_up, w2)


def _combine_body(ys_hbm, pos_hbm, w_hbm, out_hbm,
                  pos_b, w_v, ybuf_v, obuf_v, gsem):
    cc = lax.axis_index("c")
    sid = lax.axis_index("s")
    wid = sid * NC + cc
    lanes = _lane_iota()

    for c in range(4):
        pltpu.sync_copy(pos_hbm.at[4 * wid + c], pos_b[c])
    pltpu.sync_copy(w_hbm.at[pl.ds(CHUNK * wid, CHUNK)], w_v)
    tok0 = (T // 32) * wid

    for c in range(4):
        pltpu.async_copy(ys_hbm.at[pos_b[c]], ybuf_v, gsem).wait()
        for i in range(L):
            wvec = w_v[pl.ds(32 * c + L * (i // 8), L)]
            lo = (2 * i) % L
            w0 = _lane_splat(wvec, lo)
            w1s = _lane_splat(wvec, lo + 1)

            def body(j, _):
                ya = ybuf_v[2 * i, pl.ds(j * L, L)]
                yb = ybuf_v[2 * i + 1, pl.ds(j * L, L)]
                obuf_v[i, pl.ds(j * L, L)] = ya * w0 + yb * w1s
                return 0

            lax.fori_loop(0, D // L, body, 0)
        pltpu.sync_copy(obuf_v, out_hbm.at[pl.ds(tok0 + L * c, L)])


def _combine(ys, pos, wflat):
    kfn = pl.kernel(
        _combine_body,
        out_type=jax.ShapeDtypeStruct((T, D), jnp.float32),
        mesh=plsc.VectorSubcoreMesh(**_MESH),
        scratch_types=[
            tuple(pltpu.VMEM((32,), jnp.int32) for _ in range(4)),
            pltpu.VMEM((CHUNK,), jnp.float32),
            pltpu.VMEM((32, D), jnp.float32),
            pltpu.VMEM((L, D), jnp.float32),
            pltpu.SemaphoreType.DMA,
        ],
    )
    return kfn(ys, pos, wflat)


@jax.jit
def kernel(x, gate_w, w1, w1_up, w2):
    topi, topw = _router(x, gate_w)
    eids = topi.reshape(N)
    wflat = topw.reshape(N)
    pos, blkmap, valid, xs = _dispatch(eids, x)
    ys = _gemm(blkmap, valid, xs, w1, w1_up, w2)
    return _combine(ys, pos, wflat)


# speedup vs baseline: 1.2395x; 1.2395x over previous
import functools

import jax
import jax.numpy as jnp
from jax import lax
from jax.experimental import pallas as pl
from jax.experimental.pallas import tpu as pltpu
from jax.experimental.pallas import tpu_sc as plsc

T = 2048
D = 1024
F = 1408
E = 8
K = 2

N = T * K
BTG = 256
NPAD = N + E * BTG
NBLK = NPAD // BTG
NC = 2
NS = 16
L = 16
CHUNK = N // (NC * NS)
SCAN = N // NS

_MESH = dict(core_axis_name="c", subcore_axis_name="s", num_cores=NC,
             num_subcores=NS)


def _router_body(x_ref, gate_ref, topi_ref, topw_ref):
    logits = lax.dot_general(x_ref[...], gate_ref[...],
                             (((1,), (1,)), ((), ())),
                             preferred_element_type=jnp.float32)
    m = jnp.max(logits, axis=1, keepdims=True)
    ex = jnp.exp(logits - m)
    probs = ex / jnp.sum(ex, axis=1, keepdims=True)
    iota_e = lax.broadcasted_iota(jnp.int32, probs.shape, 1)
    m1 = jnp.max(probs, axis=1, keepdims=True)
    i1 = jnp.min(jnp.where(probs == m1, iota_e, E), axis=1, keepdims=True)
    masked = jnp.where(iota_e == i1, -jnp.inf, probs)
    m2 = jnp.max(masked, axis=1, keepdims=True)
    i2 = jnp.min(jnp.where(masked == m2, iota_e, E), axis=1, keepdims=True)
    topi_ref[...] = jnp.concatenate([i1, i2], axis=1)
    topw_ref[...] = jnp.concatenate([m1, m2], axis=1)


def _router(x, gate_w):
    return pl.pallas_call(
        _router_body,
        out_shape=(jax.ShapeDtypeStruct((T, K), jnp.int32),
                   jax.ShapeDtypeStruct((T, K), jnp.float32)),
    )(x, gate_w)


def _lane_iota():
    return lax.broadcasted_iota(jnp.int32, (L,), 0)


_DNUMS = lax.GatherDimensionNumbers(
    offset_dims=(), collapsed_slice_dims=(0,), start_index_map=(0,))


def _take(v, idx):
    return lax.gather(v, idx[:, None], _DNUMS, slice_sizes=(1,),
                      mode=lax.GatherScatterMode.PROMISE_IN_BOUNDS)


def _vsum(v):
    ln = _lane_iota()
    for k in (1, 2, 4, 8):
        v = v + _take(v, jnp.bitwise_xor(ln, k))
    return v


def _vcumsum(v):
    ln = _lane_iota()
    for k in (1, 2, 4, 8):
        shifted = _take(v, jnp.maximum(ln - k, 0))
        v = v + jnp.where(ln >= k, shifted, jnp.zeros_like(v))
    return v


def _lane_splat(vec, lane):
    return _vsum(jnp.where(_lane_iota() == lane, vec, jnp.zeros_like(vec)))


def _dispatch_body(eids_hbm, x_hbm, pos_hbm, blkmap_hbm, valid_hbm, xs_hbm,
                   ids_v, cnt_v, allcnt_v, pos_b, tok_b, xbuf_b,
                   shared_cnt, gsem, ssem):
    cc = lax.axis_index("c")
    sid = lax.axis_index("s")
    lanes = _lane_iota()

    pltpu.sync_copy(eids_hbm.at[pl.ds(sid * SCAN, SCAN)], ids_v)
    counts = jnp.zeros((L,), jnp.int32)
    for v in range(SCAN // L):
        ids = ids_v[pl.ds(v * L, L)]
        for e in range(E):
            pc = _vsum(jnp.where(ids == e, 1, 0))
            counts = counts + jnp.where(lanes == e, pc, 0)
    cnt_v[...] = counts
    pltpu.sync_copy(cnt_v, shared_cnt.at[pl.ds(sid * L, L)])
    plsc.subcore_barrier()

    pltpu.sync_copy(shared_cnt, allcnt_v)
    totals = jnp.zeros((L,), jnp.int32)
    prefix = jnp.zeros((L,), jnp.int32)
    for s in range(NS):
        row = allcnt_v[pl.ds(s * L, L)]
        totals = totals + row
        prefix = prefix + jnp.where(s < sid, row, 0)
    asize = jnp.where(lanes < E,
                      jnp.bitwise_and(totals + (BTG - 1), -BTG), 0)
    cum_asize = _vcumsum(asize)
    aoff = cum_asize - asize
    run = aoff + prefix
    total_pad = _lane_splat(cum_asize, E - 1)

    ends = [_lane_splat(cum_asize, e) for e in range(E)]

    @pl.when(jnp.logical_and(cc == 0, sid == 0))
    def _():
        for b3 in range(3):
            bs = (b3 * L + lanes) * BTG
            eid = jnp.zeros((L,), jnp.int32)
            for e in range(E - 1):
                eid = eid + jnp.where(bs >= ends[e], 1, 0)
            cnt_v[...] = eid
            pltpu.sync_copy(cnt_v, blkmap_hbm.at[pl.ds(b3 * L, L)])
            cnt_v[...] = jnp.where(bs < total_pad, 1, 0)
            pltpu.sync_copy(cnt_v, valid_hbm.at[pl.ds(b3 * L, L)])

    for v in range(SCAN // L):
        ids = ids_v[pl.ds(v * L, L)]
        pos = jnp.zeros((L,), jnp.int32)
        for e in range(E):
            msk = ids == e
            mi = jnp.where(msk, 1, 0)
            cum = _vcumsum(mi)
            s_e = _lane_splat(run, e)
            pos = jnp.where(msk, s_e + cum - 1, pos)
            run = run + jnp.where(lanes == e, _vsum(mi), 0)
        tok = (sid * SCAN + v * L + lanes) >> 1
        j, half = (v % 8) // 2, (v % 2) * L

        @pl.when(cc == (0 if v < 8 else 1))
        def _(pos=pos, tok=tok, j=j, half=half):
            pos_b[j][pl.ds(half, L)] = pos
            tok_b[j][pl.ds(half, L)] = tok

    rbase = NS // 2 * sid + 4 * cc
    for j in range(4):
        pltpu.sync_copy(pos_b[j], pos_hbm.at[rbase + j])
    g = [None] * 4
    g[0] = pltpu.async_copy(x_hbm.at[tok_b[0]], xbuf_b[0], gsem)
    for j in range(4):
        g[j].wait()
        if j < 3:
            g[j + 1] = pltpu.async_copy(x_hbm.at[tok_b[j + 1]],
                                        xbuf_b[(j + 1) % 2], gsem)
        pltpu.async_copy(xbuf_b[j % 2], xs_hbm.at[pos_b[j]], ssem).wait()


def _dispatch(eids, x):
    kfn = pl.kernel(
        _dispatch_body,
        out_type=(jax.ShapeDtypeStruct((N // 32, 32), jnp.int32),
                  jax.ShapeDtypeStruct((48,), jnp.int32),
                  jax.ShapeDtypeStruct((48,), jnp.int32),
                  jax.ShapeDtypeStruct((NPAD, D), jnp.float32)),
        mesh=plsc.VectorSubcoreMesh(**_MESH),
        scratch_types=[
            pltpu.VMEM((SCAN,), jnp.int32),
            pltpu.VMEM((L,), jnp.int32),
            pltpu.VMEM((NS * L,), jnp.int32),
            tuple(pltpu.VMEM((32,), jnp.int32) for _ in range(4)),
            tuple(pltpu.VMEM((32,), jnp.int32) for _ in range(4)),
            tuple(pltpu.VMEM((32, D), jnp.float32) for _ in range(2)),
            pltpu.VMEM_SHARED((NS * L,), jnp.int32),
            pltpu.SemaphoreType.DMA,
            pltpu.SemaphoreType.DMA,
        ],
    )
    return kfn(eids, x)


def _gemm_body(bm_ref, valid_ref, xs_ref, w1_ref, w1u_ref, w2_ref, y_ref):
    s = pl.program_id(0)

    @pl.when(valid_ref[s] != 0)
    def _():
        xb = xs_ref[...]
        h = lax.dot_general(xb, w1_ref[0], (((1,), (1,)), ((), ())),
                            preferred_element_type=jnp.float32)
        u = lax.dot_general(xb, w1u_ref[0], (((1,), (1,)), ((), ())),
                            preferred_element_type=jnp.float32)
        g = h * (1.0 / (1.0 + jnp.exp(-h))) * u
        y_ref[...] = lax.dot_general(g, w2_ref[0], (((1,), (1,)), ((), ())),
                                     preferred_element_type=jnp.float32)


def _gemm(blkmap, valid, xs, w1, w1_up, w2):
    grid_spec = pltpu.PrefetchScalarGridSpec(
        num_scalar_prefetch=2,
        grid=(NBLK,),
        in_specs=[
            pl.BlockSpec((BTG, D), lambda s, bm, vd: (s, 0)),
            pl.BlockSpec((1, F, D), lambda s, bm, vd: (bm[s], 0, 0)),
            pl.BlockSpec((1, F, D), lambda s, bm, vd: (bm[s], 0, 0)),
            pl.BlockSpec((1, D, F), lambda s, bm, vd: (bm[s], 0, 0)),
        ],
        out_specs=pl.BlockSpec((BTG, D), lambda s, bm, vd: (s, 0)),
    )
    return pl.pallas_call(
        _gemm_body,
        grid_spec=grid_spec,
        out_shape=jax.ShapeDtypeStruct((NPAD, D), jnp.float32),
        compiler_params=pltpu.CompilerParams(
            dimension_semantics=("arbitrary",),
        ),
    )(blkmap, valid, xs, w1, w1_up, w2)


def _combine_body(ys_hbm, pos_hbm, w_hbm, out_hbm,
                  pos_b, w_v, ybuf_b, obuf_v, gsem):
    cc = lax.axis_index("c")
    sid = lax.axis_index("s")
    wid = sid * NC + cc
    lanes = _lane_iota()

    for c in range(4):
        pltpu.sync_copy(pos_hbm.at[4 * wid + c], pos_b[c])
    pltpu.sync_copy(w_hbm.at[pl.ds(CHUNK * wid, CHUNK)], w_v)
    tok0 = (T // 32) * wid

    gd = [None] * 4
    gd[0] = pltpu.async_copy(ys_hbm.at[pos_b[0]], ybuf_b[0], gsem)
    for c in range(4):
        gd[c].wait()
        if c < 3:
            gd[c + 1] = pltpu.async_copy(ys_hbm.at[pos_b[c + 1]],
                                         ybuf_b[(c + 1) % 2], gsem)
        ybuf_v = ybuf_b[c % 2]
        for i in range(L):
            wvec = w_v[pl.ds(32 * c + L * (i // 8), L)]
            lo = (2 * i) % L
            w0 = _lane_splat(wvec, lo)
            w1s = _lane_splat(wvec, lo + 1)

            def body(j, _):
                for q in range(4):
                    sl = pl.ds(j * (4 * L) + q * L, L)
                    ya = ybuf_v[2 * i, sl]
                    yb = ybuf_v[2 * i + 1, sl]
                    obuf_v[i, sl] = ya * w0 + yb * w1s
                return 0

            lax.fori_loop(0, D // (4 * L), body, 0)
        pltpu.sync_copy(obuf_v, out_hbm.at[pl.ds(tok0 + L * c, L)])


def _combine(ys, pos, wflat):
    kfn = pl.kernel(
        _combine_body,
        out_type=jax.ShapeDtypeStruct((T, D), jnp.float32),
        mesh=plsc.VectorSubcoreMesh(**_MESH),
        scratch_types=[
            tuple(pltpu.VMEM((32,), jnp.int32) for _ in range(4)),
            pltpu.VMEM((CHUNK,), jnp.float32),
            tuple(pltpu.VMEM((32, D), jnp.float32) for _ in range(2)),
            pltpu.VMEM((L, D), jnp.float32),
            pltpu.SemaphoreType.DMA,
        ],
    )
    return kfn(ys, pos, wflat)


@jax.jit
def kernel(x, gate_w, w1, w1_up, w2):
    topi, topw = _router(x, gate_w)
    eids = topi.reshape(N)
    wflat = topw.reshape(N)
    pos, blkmap, valid, xs = _dispatch(eids, x)
    ys = _gemm(blkmap, valid, xs, w1, w1_up, w2)
    return _combine(ys, pos, wflat)


# speedup vs baseline: 1.3321x; 1.0747x over previous
import functools

import jax
import jax.numpy as jnp
from jax import lax
from jax.experimental import pallas as pl
from jax.experimental.pallas import tpu as pltpu
from jax.experimental.pallas import tpu_sc as plsc

T = 2048
D = 1024
F = 1408
E = 8
K = 2

N = T * K
BTG = 512
NPAD = N + E * BTG
NBLK = NPAD // BTG
NC = 2
NS = 16
L = 16
CHUNK = N // (NC * NS)
SCAN = N // NS

_MESH = dict(core_axis_name="c", subcore_axis_name="s", num_cores=NC,
             num_subcores=NS)


def _router_body(x_ref, gate_ref, topi_ref, topw_ref):
    logits = lax.dot_general(x_ref[...], gate_ref[...],
                             (((1,), (1,)), ((), ())),
                             preferred_element_type=jnp.float32)
    m = jnp.max(logits, axis=1, keepdims=True)
    ex = jnp.exp(logits - m)
    probs = ex / jnp.sum(ex, axis=1, keepdims=True)
    iota_e = lax.broadcasted_iota(jnp.int32, probs.shape, 1)
    m1 = jnp.max(probs, axis=1, keepdims=True)
    i1 = jnp.min(jnp.where(probs == m1, iota_e, E), axis=1, keepdims=True)
    masked = jnp.where(iota_e == i1, -jnp.inf, probs)
    m2 = jnp.max(masked, axis=1, keepdims=True)
    i2 = jnp.min(jnp.where(masked == m2, iota_e, E), axis=1, keepdims=True)
    topi_ref[...] = jnp.concatenate([i1, i2], axis=1)
    topw_ref[...] = jnp.concatenate([m1, m2], axis=1)


def _router(x, gate_w):
    return pl.pallas_call(
        _router_body,
        out_shape=(jax.ShapeDtypeStruct((T, K), jnp.int32),
                   jax.ShapeDtypeStruct((T, K), jnp.float32)),
    )(x, gate_w)


def _lane_iota():
    return lax.broadcasted_iota(jnp.int32, (L,), 0)


_DNUMS = lax.GatherDimensionNumbers(
    offset_dims=(), collapsed_slice_dims=(0,), start_index_map=(0,))


def _take(v, idx):
    return lax.gather(v, idx[:, None], _DNUMS, slice_sizes=(1,),
                      mode=lax.GatherScatterMode.PROMISE_IN_BOUNDS)


def _vsum(v):
    ln = _lane_iota()
    for k in (1, 2, 4, 8):
        v = v + _take(v, jnp.bitwise_xor(ln, k))
    return v


def _vcumsum(v):
    ln = _lane_iota()
    for k in (1, 2, 4, 8):
        shifted = _take(v, jnp.maximum(ln - k, 0))
        v = v + jnp.where(ln >= k, shifted, jnp.zeros_like(v))
    return v


def _lane_splat(vec, lane):
    return _vsum(jnp.where(_lane_iota() == lane, vec, jnp.zeros_like(vec)))


def _dispatch_body(eids_hbm, x_hbm, pos_hbm, blkmap_hbm, valid_hbm, xs_hbm,
                   ids_v, cnt_v, allcnt_v, pos_b, tok_b, xbuf_b,
                   shared_cnt, gsem, ssem):
    cc = lax.axis_index("c")
    sid = lax.axis_index("s")
    lanes = _lane_iota()

    pltpu.sync_copy(eids_hbm.at[pl.ds(sid * SCAN, SCAN)], ids_v)
    counts = jnp.zeros((L,), jnp.int32)
    for v in range(SCAN // L):
        ids = ids_v[pl.ds(v * L, L)]
        for e in range(E):
            pc = _vsum(jnp.where(ids == e, 1, 0))
            counts = counts + jnp.where(lanes == e, pc, 0)
    cnt_v[...] = counts
    pltpu.sync_copy(cnt_v, shared_cnt.at[pl.ds(sid * L, L)])
    plsc.subcore_barrier()

    pltpu.sync_copy(shared_cnt, allcnt_v)
    totals = jnp.zeros((L,), jnp.int32)
    prefix = jnp.zeros((L,), jnp.int32)
    for s in range(NS):
        row = allcnt_v[pl.ds(s * L, L)]
        totals = totals + row
        prefix = prefix + jnp.where(s < sid, row, 0)
    asize = jnp.where(lanes < E,
                      jnp.bitwise_and(totals + (BTG - 1), -BTG), 0)
    cum_asize = _vcumsum(asize)
    aoff = cum_asize - asize
    run = aoff + prefix
    total_pad = _lane_splat(cum_asize, E - 1)

    ends = [_lane_splat(cum_asize, e) for e in range(E)]

    @pl.when(jnp.logical_and(cc == 0, sid == 0))
    def _():
        for b3 in range(3):
            bs = (b3 * L + lanes) * BTG
            eid = jnp.zeros((L,), jnp.int32)
            for e in range(E - 1):
                eid = eid + jnp.where(bs >= ends[e], 1, 0)
            cnt_v[...] = eid
            pltpu.sync_copy(cnt_v, blkmap_hbm.at[pl.ds(b3 * L, L)])
            cnt_v[...] = jnp.where(bs < total_pad, 1, 0)
            pltpu.sync_copy(cnt_v, valid_hbm.at[pl.ds(b3 * L, L)])

    for v in range(SCAN // L):
        ids = ids_v[pl.ds(v * L, L)]
        pos = jnp.zeros((L,), jnp.int32)
        for e in range(E):
            msk = ids == e
            mi = jnp.where(msk, 1, 0)
            cum = _vcumsum(mi)
            s_e = _lane_splat(run, e)
            pos = jnp.where(msk, s_e + cum - 1, pos)
            run = run + jnp.where(lanes == e, _vsum(mi), 0)
        tok = (sid * SCAN + v * L + lanes) >> 1
        j, half = (v % 8) // 2, (v % 2) * L

        @pl.when(cc == (0 if v < 8 else 1))
        def _(pos=pos, tok=tok, j=j, half=half):
            pos_b[j][pl.ds(half, L)] = pos
            tok_b[j][pl.ds(half, L)] = tok

    rbase = NS // 2 * sid + 4 * cc
    for j in range(4):
        pltpu.sync_copy(pos_b[j], pos_hbm.at[rbase + j])
    g = [None] * 4
    g[0] = pltpu.async_copy(x_hbm.at[tok_b[0]], xbuf_b[0], gsem)
    for j in range(4):
        g[j].wait()
        if j < 3:
            g[j + 1] = pltpu.async_copy(x_hbm.at[tok_b[j + 1]],
                                        xbuf_b[(j + 1) % 2], gsem)
        pltpu.async_copy(xbuf_b[j % 2], xs_hbm.at[pos_b[j]], ssem).wait()


def _dispatch(eids, x):
    kfn = pl.kernel(
        _dispatch_body,
        out_type=(jax.ShapeDtypeStruct((N // 32, 32), jnp.int32),
                  jax.ShapeDtypeStruct((48,), jnp.int32),
                  jax.ShapeDtypeStruct((48,), jnp.int32),
                  jax.ShapeDtypeStruct((NPAD, D), jnp.float32)),
        mesh=plsc.VectorSubcoreMesh(**_MESH),
        scratch_types=[
            pltpu.VMEM((SCAN,), jnp.int32),
            pltpu.VMEM((L,), jnp.int32),
            pltpu.VMEM((NS * L,), jnp.int32),
            tuple(pltpu.VMEM((32,), jnp.int32) for _ in range(4)),
            tuple(pltpu.VMEM((32,), jnp.int32) for _ in range(4)),
            tuple(pltpu.VMEM((32, D), jnp.float32) for _ in range(2)),
            pltpu.VMEM_SHARED((NS * L,), jnp.int32),
            pltpu.SemaphoreType.DMA,
            pltpu.SemaphoreType.DMA,
        ],
    )
    return kfn(eids, x)


def _gemm_body(bm_ref, valid_ref, xs_ref, w1_ref, w1u_ref, w2_ref, y_ref):
    s = pl.program_id(0)

    @pl.when(valid_ref[s] != 0)
    def _():
        xb = xs_ref[...]
        h = lax.dot_general(xb, w1_ref[0], (((1,), (1,)), ((), ())),
                            preferred_element_type=jnp.float32)
        u = lax.dot_general(xb, w1u_ref[0], (((1,), (1,)), ((), ())),
                            preferred_element_type=jnp.float32)
        g = h * (1.0 / (1.0 + jnp.exp(-h))) * u
        y_ref[...] = lax.dot_general(g, w2_ref[0], (((1,), (1,)), ((), ())),
                                     preferred_element_type=jnp.float32)


def _gemm(blkmap, valid, xs, w1, w1_up, w2):
    grid_spec = pltpu.PrefetchScalarGridSpec(
        num_scalar_prefetch=2,
        grid=(NBLK,),
        in_specs=[
            pl.BlockSpec((BTG, D), lambda s, bm, vd: (s, 0)),
            pl.BlockSpec((1, F, D), lambda s, bm, vd: (bm[s], 0, 0)),
            pl.BlockSpec((1, F, D), lambda s, bm, vd: (bm[s], 0, 0)),
            pl.BlockSpec((1, D, F), lambda s, bm, vd: (bm[s], 0, 0)),
        ],
        out_specs=pl.BlockSpec((BTG, D), lambda s, bm, vd: (s, 0)),
    )
    return pl.pallas_call(
        _gemm_body,
        grid_spec=grid_spec,
        out_shape=jax.ShapeDtypeStruct((NPAD, D), jnp.float32),
        compiler_params=pltpu.CompilerParams(
            dimension_semantics=("arbitrary",),
        ),
    )(blkmap, valid, xs, w1, w1_up, w2)


def _combine_body(ys_hbm, pos_hbm, w_hbm, out_hbm,
                  pos_b, w_v, ybuf_b, obuf_v, gsem):
    cc = lax.axis_index("c")
    sid = lax.axis_index("s")
    wid = sid * NC + cc
    lanes = _lane_iota()

    for c in range(4):
        pltpu.sync_copy(pos_hbm.at[4 * wid + c], pos_b[c])
    pltpu.sync_copy(w_hbm.at[pl.ds(CHUNK * wid, CHUNK)], w_v)
    tok0 = (T // 32) * wid

    gd = [None] * 4
    gd[0] = pltpu.async_copy(ys_hbm.at[pos_b[0]], ybuf_b[0], gsem)
    for c in range(4):
        gd[c].wait()
        if c < 3:
            gd[c + 1] = pltpu.async_copy(ys_hbm.at[pos_b[c + 1]],
                                         ybuf_b[(c + 1) % 2], gsem)
        ybuf_v = ybuf_b[c % 2]
        for i in range(L):
            wvec = w_v[pl.ds(32 * c + L * (i // 8), L)]
            lo = (2 * i) % L
            w0 = _lane_splat(wvec, lo)
            w1s = _lane_splat(wvec, lo + 1)

            def body(j, _):
                for q in range(4):
                    sl = pl.ds(j * (4 * L) + q * L, L)
                    ya = ybuf_v[2 * i, sl]
                    yb = ybuf_v[2 * i + 1, sl]
                    obuf_v[i, sl] = ya * w0 + yb * w1s
                return 0

            lax.fori_loop(0, D // (4 * L), body, 0)
        pltpu.sync_copy(obuf_v, out_hbm.at[pl.ds(tok0 + L * c, L)])


def _combine(ys, pos, wflat):
    kfn = pl.kernel(
        _combine_body,
        out_type=jax.ShapeDtypeStruct((T, D), jnp.float32),
        mesh=plsc.VectorSubcoreMesh(**_MESH),
        scratch_types=[
            tuple(pltpu.VMEM((32,), jnp.int32) for _ in range(4)),
            pltpu.VMEM((CHUNK,), jnp.float32),
            tuple(pltpu.VMEM((32, D), jnp.float32) for _ in range(2)),
            pltpu.VMEM((L, D), jnp.float32),
            pltpu.SemaphoreType.DMA,
        ],
    )
    return kfn(ys, pos, wflat)


@jax.jit
def kernel(x, gate_w, w1, w1_up, w2):
    topi, topw = _router(x, gate_w)
    eids = topi.reshape(N)
    wflat = topw.reshape(N)
    pos, blkmap, valid, xs = _dispatch(eids, x)
    ys = _gemm(blkmap, valid, xs, w1, w1_up, w2)
    return _combine(ys, pos, wflat)


# speedup vs baseline: 1.5147x; 1.1371x over previous
import functools

import jax
import jax.numpy as jnp
from jax import lax
from jax.experimental import pallas as pl
from jax.experimental.pallas import tpu as pltpu
from jax.experimental.pallas import tpu_sc as plsc

T = 2048
D = 1024
F = 1408
E = 8
K = 2

N = T * K
BTG = 512
NPAD = N + E * BTG
NBLK = NPAD // BTG
NC = 2
NS = 16
L = 16
CHUNK = N // (NC * NS)
SCAN = N // NS

_MESH = dict(core_axis_name="c", subcore_axis_name="s", num_cores=NC,
             num_subcores=NS)


def _router_body(x_ref, gate_ref, topi_ref, topw_ref):
    logits = lax.dot_general(x_ref[...], gate_ref[...],
                             (((1,), (1,)), ((), ())),
                             preferred_element_type=jnp.float32)
    m = jnp.max(logits, axis=1, keepdims=True)
    ex = jnp.exp(logits - m)
    probs = ex / jnp.sum(ex, axis=1, keepdims=True)
    iota_e = lax.broadcasted_iota(jnp.int32, probs.shape, 1)
    m1 = jnp.max(probs, axis=1, keepdims=True)
    i1 = jnp.min(jnp.where(probs == m1, iota_e, E), axis=1, keepdims=True)
    masked = jnp.where(iota_e == i1, -jnp.inf, probs)
    m2 = jnp.max(masked, axis=1, keepdims=True)
    i2 = jnp.min(jnp.where(masked == m2, iota_e, E), axis=1, keepdims=True)
    topi_ref[...] = jnp.concatenate([i1, i2], axis=1)
    topw_ref[...] = jnp.concatenate([m1, m2], axis=1)


def _router(x, gate_w):
    return pl.pallas_call(
        _router_body,
        out_shape=(jax.ShapeDtypeStruct((T, K), jnp.int32),
                   jax.ShapeDtypeStruct((T, K), jnp.float32)),
    )(x, gate_w)


def _lane_iota():
    return lax.broadcasted_iota(jnp.int32, (L,), 0)


_DNUMS = lax.GatherDimensionNumbers(
    offset_dims=(), collapsed_slice_dims=(0,), start_index_map=(0,))


def _take(v, idx):
    return lax.gather(v, idx[:, None], _DNUMS, slice_sizes=(1,),
                      mode=lax.GatherScatterMode.PROMISE_IN_BOUNDS)


def _vsum(v):
    ln = _lane_iota()
    for k in (1, 2, 4, 8):
        v = v + _take(v, jnp.bitwise_xor(ln, k))
    return v


def _vcumsum(v):
    ln = _lane_iota()
    for k in (1, 2, 4, 8):
        shifted = _take(v, jnp.maximum(ln - k, 0))
        v = v + jnp.where(ln >= k, shifted, jnp.zeros_like(v))
    return v


def _lane_splat(vec, lane):
    return _vsum(jnp.where(_lane_iota() == lane, vec, jnp.zeros_like(vec)))


def _dispatch_body(eids_hbm, x_hbm, pos_hbm, blkmap_hbm, valid_hbm,
                   xs_hbm,
                   ids_v, cnt_v, allcnt_v, pos_b, tok_b, xbuf_b,
                   shared_cnt, gsem, ssem):
    cc = lax.axis_index("c")
    sid = lax.axis_index("s")
    lanes = _lane_iota()

    pltpu.sync_copy(eids_hbm.at[pl.ds(sid * SCAN, SCAN)], ids_v)
    counts = jnp.zeros((L,), jnp.int32)
    for v in range(SCAN // L):
        ids = ids_v[pl.ds(v * L, L)]
        for e in range(E):
            pc = _vsum(jnp.where(ids == e, 1, 0))
            counts = counts + jnp.where(lanes == e, pc, 0)
    cnt_v[...] = counts
    pltpu.sync_copy(cnt_v, shared_cnt.at[pl.ds(sid * L, L)])
    plsc.subcore_barrier()

    pltpu.sync_copy(shared_cnt, allcnt_v)
    totals = jnp.zeros((L,), jnp.int32)
    prefix = jnp.zeros((L,), jnp.int32)
    for s in range(NS):
        row = allcnt_v[pl.ds(s * L, L)]
        totals = totals + row
        prefix = prefix + jnp.where(s < sid, row, 0)
    asize = jnp.where(lanes < E,
                      jnp.bitwise_and(totals + (BTG - 1), -BTG), 0)
    cum_asize = _vcumsum(asize)
    aoff = cum_asize - asize
    run = aoff + prefix
    total_pad = _lane_splat(cum_asize, E - 1)

    ends = [_lane_splat(cum_asize, e) for e in range(E)]

    @pl.when(jnp.logical_and(cc == 0, sid == 0))
    def _():
        for b3 in range(3):
            bs = (b3 * L + lanes) * BTG
            eid = jnp.zeros((L,), jnp.int32)
            for e in range(E - 1):
                eid = eid + jnp.where(bs >= ends[e], 1, 0)
            cnt_v[...] = eid
            pltpu.sync_copy(cnt_v, blkmap_hbm.at[pl.ds(b3 * L, L)])
            cnt_v[...] = jnp.where(bs < total_pad, 1, 0)
            pltpu.sync_copy(cnt_v, valid_hbm.at[pl.ds(b3 * L, L)])

    for v in range(SCAN // L):
        ids = ids_v[pl.ds(v * L, L)]
        pos = jnp.zeros((L,), jnp.int32)
        for e in range(E):
            msk = ids == e
            mi = jnp.where(msk, 1, 0)
            cum = _vcumsum(mi)
            s_e = _lane_splat(run, e)
            pos = jnp.where(msk, s_e + cum - 1, pos)
            run = run + jnp.where(lanes == e, _vsum(mi), 0)
        tok = (sid * SCAN + v * L + lanes) >> 1
        j, half = (v % 8) // 2, (v % 2) * L

        @pl.when(cc == (0 if v < 8 else 1))
        def _(pos=pos, tok=tok, j=j, half=half):
            pos_b[j][pl.ds(half, L)] = pos
            tok_b[j][pl.ds(half, L)] = tok

    rbase = NS // 2 * sid + 4 * cc
    for j in range(4):
        pltpu.sync_copy(pos_b[j], pos_hbm.at[rbase + j])
    g = [None] * 4
    g[0] = pltpu.async_copy(x_hbm.at[tok_b[0]], xbuf_b[0], gsem)
    for j in range(4):
        g[j].wait()
        if j < 3:
            g[j + 1] = pltpu.async_copy(x_hbm.at[tok_b[j + 1]],
                                        xbuf_b[(j + 1) % 2], gsem)
        pltpu.async_copy(xbuf_b[j % 2], xs_hbm.at[pos_b[j]], ssem).wait()


def _dispatch(eids, x):
    kfn = pl.kernel(
        _dispatch_body,
        out_type=(jax.ShapeDtypeStruct((N // 32, 32), jnp.int32),
                  jax.ShapeDtypeStruct((48,), jnp.int32),
                  jax.ShapeDtypeStruct((48,), jnp.int32),
                  jax.ShapeDtypeStruct((NPAD, D), jnp.float32)),
        mesh=plsc.VectorSubcoreMesh(**_MESH),
        scratch_types=[
            pltpu.VMEM((SCAN,), jnp.int32),
            pltpu.VMEM((L,), jnp.int32),
            pltpu.VMEM((NS * L,), jnp.int32),
            tuple(pltpu.VMEM((32,), jnp.int32) for _ in range(4)),
            tuple(pltpu.VMEM((32,), jnp.int32) for _ in range(4)),
            tuple(pltpu.VMEM((32, D), jnp.float32) for _ in range(2)),
            pltpu.VMEM_SHARED((NS * L,), jnp.int32),
            pltpu.SemaphoreType.DMA,
            pltpu.SemaphoreType.DMA,
        ],
    )
    return kfn(eids, x)


def _gemm_body(bm_ref, valid_ref, xs_ref, w1_ref, w1u_ref, w2_ref, y_ref):
    s = pl.program_id(0)

    @pl.when(valid_ref[s] != 0)
    def _():
        xb = xs_ref[...]
        h = lax.dot_general(xb, w1_ref[0], (((1,), (1,)), ((), ())),
                            preferred_element_type=jnp.float32)
        u = lax.dot_general(xb, w1u_ref[0], (((1,), (1,)), ((), ())),
                            preferred_element_type=jnp.float32)
        g = h * (1.0 / (1.0 + jnp.exp(-h))) * u
        y_ref[...] = lax.dot_general(g, w2_ref[0], (((1,), (1,)), ((), ())),
                                     preferred_element_type=jnp.float32)


def _gemm(blkmap, valid, xs, w1, w1_up, w2):
    grid_spec = pltpu.PrefetchScalarGridSpec(
        num_scalar_prefetch=2,
        grid=(NBLK,),
        in_specs=[
            pl.BlockSpec((BTG, D), lambda s, bm, vd: (s, 0)),
            pl.BlockSpec((1, F, D), lambda s, bm, vd: (bm[s], 0, 0)),
            pl.BlockSpec((1, F, D), lambda s, bm, vd: (bm[s], 0, 0)),
            pl.BlockSpec((1, D, F), lambda s, bm, vd: (bm[s], 0, 0)),
        ],
        out_specs=pl.BlockSpec((BTG, D), lambda s, bm, vd: (s, 0)),
    )
    return pl.pallas_call(
        _gemm_body,
        grid_spec=grid_spec,
        out_shape=jax.ShapeDtypeStruct((NPAD, D), jnp.float32),
        compiler_params=pltpu.CompilerParams(
            dimension_semantics=("arbitrary",),
        ),
    )(blkmap, valid, xs, w1, w1_up, w2)


def _combine_body(ys_hbm, pos_hbm, w_hbm, out_hbm,
                  pos_b, w_v, ybuf_b, obuf_v, gsem):
    cc = lax.axis_index("c")
    sid = lax.axis_index("s")
    wid = sid * NC + cc

    for c in range(4):
        pltpu.sync_copy(pos_hbm.at[4 * wid + c], pos_b[c])
    pltpu.sync_copy(w_hbm.at[pl.ds(CHUNK * wid, CHUNK)], w_v)
    tok0 = (T // 32) * wid

    gd = [None] * 4
    gd[0] = pltpu.async_copy(ys_hbm.at[pos_b[0]], ybuf_b[0], gsem)
    for c in range(4):
        gd[c].wait()
        if c < 3:
            gd[c + 1] = pltpu.async_copy(ys_hbm.at[pos_b[c + 1]],
                                         ybuf_b[(c + 1) % 2], gsem)
        ybuf_v = ybuf_b[c % 2]
        for i in range(L):
            wvec = w_v[pl.ds(32 * c + L * (i // 8), L)]
            lo = (2 * i) % L
            w0 = _lane_splat(wvec, lo)
            w1s = _lane_splat(wvec, lo + 1)

            def body(j, _):
                for q in range(8):
                    sl = pl.ds(j * (8 * L) + q * L, L)
                    obuf_v[i, sl] = (ybuf_v[2 * i, sl] * w0
                                     + ybuf_v[2 * i + 1, sl] * w1s)
                return 0

            lax.fori_loop(0, D // (8 * L), body, 0)
        pltpu.sync_copy(obuf_v, out_hbm.at[pl.ds(tok0 + L * c, L)])


def _combine(ys, pos, wflat):
    kfn = pl.kernel(
        _combine_body,
        out_type=jax.ShapeDtypeStruct((T, D), jnp.float32),
        mesh=plsc.VectorSubcoreMesh(**_MESH),
        scratch_types=[
            tuple(pltpu.VMEM((32,), jnp.int32) for _ in range(4)),
            pltpu.VMEM((CHUNK,), jnp.float32),
            tuple(pltpu.VMEM((32, D), jnp.float32) for _ in range(2)),
            pltpu.VMEM((L, D), jnp.float32),
            pltpu.SemaphoreType.DMA,
        ],
    )
    return kfn(ys, pos, wflat)


@jax.jit
def kernel(x, gate_w, w1, w1_up, w2):
    topi, topw = _router(x, gate_w)
    eids = topi.reshape(N)
    wflat = topw.reshape(N)
    pos, blkmap, valid, xs = _dispatch(eids, x)
    ys = _gemm(blkmap, valid, xs, w1, w1_up, w2)
    return _combine(ys, pos, wflat)
